# baseline (speedup 1.0000x reference)
import functools

import jax
import jax.numpy as jnp
from jax import lax
from jax.experimental import pallas as pl
from jax.experimental.pallas import tpu as pltpu
from jax.experimental.pallas import tpu_sc as plsc

N_TAG = 1000
EMB_DIM = 64
BATCH = 16384

_SC_INFO = plsc.get_sparse_core_info()
_NC = _SC_INFO.num_cores
_NS = _SC_INFO.num_subcores
_NW = _NC * _NS
_BPW = BATCH // _NW


def _sc_gather_body(tag_tbl, ques_tbl, tag_ids, ques_ids, out, idx_v, rows_v, sem):
    wid = lax.axis_index("s") * _NC + lax.axis_index("c")
    base = wid * _BPW

    pltpu.sync_copy(tag_ids.at[pl.ds(base, _BPW)], idx_v)
    pltpu.async_copy(tag_tbl.at[idx_v], rows_v, sem).wait()
    pltpu.sync_copy(rows_v, out.at[pl.ds(base, _BPW)])

    pltpu.sync_copy(ques_ids.at[pl.ds(base, _BPW)], idx_v)

    def _shift(i):
        idx_v[pl.ds(i * 16, 16)] = idx_v[pl.ds(i * 16, 16)] - N_TAG

    pl.loop(0, _BPW // 16)(_shift)
    pltpu.async_copy(ques_tbl.at[idx_v], rows_v, sem).wait()
    pltpu.sync_copy(rows_v, out.at[pl.ds(BATCH + base, _BPW)])


_sc_gather = functools.partial(
    pl.kernel,
    mesh=plsc.VectorSubcoreMesh(core_axis_name="c", subcore_axis_name="s"),
    out_type=jax.ShapeDtypeStruct((2 * BATCH, EMB_DIM), jnp.float32),
    scratch_types=[
        pltpu.VMEM((_BPW,), jnp.int32),
        pltpu.VMEM((_BPW, EMB_DIM), jnp.float32),
        pltpu.SemaphoreType.DMA,
    ],
    compiler_params=pltpu.CompilerParams(use_tc_tiling_on_sc=False),
)(_sc_gather_body)


_ROWS = 256


def _fc_softmax_body(z_ref, w_ref, b_ref, o_ref):
    logits = lax.dot_general(
        z_ref[...], w_ref[...],
        (((1,), (1,)), ((), ())),
        preferred_element_type=jnp.float32,
    ) + b_ref[...]
    m = jnp.max(logits, axis=-1, keepdims=True)
    e = jnp.exp(logits - m)
    o_ref[...] = e / jnp.sum(e, axis=-1, keepdims=True)


def _fc_softmax(z, fc_w, fc_b2):
    n_rows = z.shape[0]
    return pl.pallas_call(
        _fc_softmax_body,
        grid=(n_rows // _ROWS,),
        in_specs=[
            pl.BlockSpec((_ROWS, EMB_DIM), lambda i: (i, 0)),
            pl.BlockSpec((N_TAG, EMB_DIM), lambda i: (0, 0)),
            pl.BlockSpec((1, N_TAG), lambda i: (0, 0)),
        ],
        out_specs=pl.BlockSpec((_ROWS, N_TAG), lambda i: (i, 0)),
        out_shape=jax.ShapeDtypeStruct((n_rows, N_TAG), jnp.float32),
    )(z, fc_w, fc_b2)


def kernel(tag_ids, ques_ids, tag_table, ques_table, fc_w, fc_b):
    tag_ids = tag_ids.astype(jnp.int32)
    ques_ids = ques_ids.astype(jnp.int32)
    z = _sc_gather(tag_table, ques_table, tag_ids, ques_ids)
    return _fc_softmax(z, fc_w, fc_b.reshape(1, N_TAG))
